# baseline (speedup 1.0000x reference)
import functools
import numpy as np
import jax
import jax.numpy as jnp
from jax import lax
from jax.experimental import pallas as pl
from jax.experimental.pallas import tpu as pltpu
from jax.experimental.pallas import tpu_sc as plsc

_IDX = [162, 1098, 1377]

ROWS = 16384
COLS = 2048
NC, NS = 2, 16
NW = NC * NS
ROWS_PER_W = ROWS // NW
CHUNK = 8
NCHUNKS = ROWS_PER_W // CHUNK
NBUF = 4


def _sc_body(img_hbm, out_hbm, buf, sems):
    wid = lax.axis_index("s") * NC + lax.axis_index("c")
    base = wid * ROWS_PER_W
    lane = lax.iota(jnp.int32, 16)

    def load(g, slot):
        return pltpu.make_async_copy(
            img_hbm.at[pl.ds(base + g * CHUNK, CHUNK), :],
            buf.at[slot],
            sems.at[slot],
        )

    def store(g, slot):
        return pltpu.make_async_copy(
            buf.at[slot],
            out_hbm.at[pl.ds(base + g * CHUNK, CHUNK), :],
            sems.at[NBUF + slot],
        )

    for b in range(NBUF):
        load(b, b).start()

    def outer(i, carry):
        for b in range(NBUF):
            g = i * NBUF + b
            load(g, b).wait()
            for c in _IDX:
                c0 = (c // 16) * 16
                off = c % 16
                for r in range(CHUNK):
                    v = buf[b, r, pl.ds(c0, 16)]
                    buf[b, r, pl.ds(c0, 16)] = jnp.where(
                        lane == off, jnp.float32(0.0), v
                    )
            store(g, b).start()

            @pl.when(g + NBUF < NCHUNKS)
            def _():
                store(g, b).wait()
                load(g + NBUF, b).start()

        return carry

    lax.fori_loop(0, NCHUNKS // NBUF, outer, 0)

    for b in range(NBUF):
        store(NCHUNKS - NBUF + b, b).wait()


def kernel(img):
    mesh = plsc.VectorSubcoreMesh(core_axis_name="c", subcore_axis_name="s")
    k = functools.partial(
        pl.kernel,
        mesh=mesh,
        out_type=jax.ShapeDtypeStruct((ROWS, COLS), jnp.float32),
        scratch_types=[
            pltpu.VMEM((NBUF, CHUNK, COLS), jnp.float32),
            pltpu.SemaphoreType.DMA((2 * NBUF,)),
        ],
    )(_sc_body)
    return k(img)
